# Initial kernel scaffold; baseline (speedup 1.0000x reference)
import jax
import jax.numpy as jnp
from jax.experimental import pallas as pl


def kernel(x_path, prototype, W3, b3, W2, b2, Wr, br, Wc, bc):
    raise NotImplementedError("write your pallas kernel here")


# speedup vs baseline: 1.2055x; 1.2055x over previous
import functools

import jax
import jax.numpy as jnp
from jax import lax
from jax.experimental import pallas as pl
from jax.experimental.pallas import tpu as pltpu

N, D, H, C, K = 32768, 2048, 256, 16, 64
TOPK = 10
ROWS_PER_BLOCK = 1024
NUM_BLOCKS = N // ROWS_PER_BLOCK


def _score_body(x_ref, w_ref, out_ref):
    out_ref[...] = jnp.sum(x_ref[...] * w_ref[...][None, :], axis=1)


def _scores(x_path, w):
    return pl.pallas_call(
        _score_body,
        grid=(NUM_BLOCKS,),
        in_specs=[
            pl.BlockSpec((ROWS_PER_BLOCK, D), lambda i: (i, 0)),
            pl.BlockSpec((D,), lambda i: (0,)),
        ],
        out_specs=pl.BlockSpec((ROWS_PER_BLOCK,), lambda i: (i,)),
        out_shape=jax.ShapeDtypeStruct((N,), jnp.float32),
    )(x_path, w)


def _tail_body(scores_ref, x_hbm, proto_ref, w2_ref, b2_ref, wr_ref, br_ref,
               wc_ref, bc_ref, bag_ref, prob_ref, yhat_ref, sim_ref,
               m_scratch, sem):
    s = scores_ref[...].reshape(N // 128, 128)
    rows = lax.broadcasted_iota(jnp.int32, (N // 128, 128), 0)
    cols = lax.broadcasted_iota(jnp.int32, (N // 128, 128), 1)
    lin = rows * 128 + cols

    copies = []
    for t in range(TOPK):
        m = jnp.max(s)
        idx = jnp.min(jnp.where(s == m, lin, jnp.int32(N)))
        cp = pltpu.make_async_copy(
            x_hbm.at[pl.ds(idx, 1), :], m_scratch.at[pl.ds(t, 1), :], sem)
        cp.start()
        copies.append(cp)
        s = jnp.where(lin == idx, -jnp.inf, s)
    for cp in copies:
        cp.wait()

    mrows = m_scratch[...]
    dn = (((1,), (1,)), ((), ()))
    f = lax.dot_general(mrows, w2_ref[...], dn,
                        preferred_element_type=jnp.float32) + b2_ref[...][None, :]
    p = lax.dot_general(proto_ref[...], w2_ref[...], dn,
                        preferred_element_type=jnp.float32) + b2_ref[...][None, :]

    sim_rows = []
    for t in range(TOPK):
        d = f[t:t + 1, :] - p + 1e-6
        sim_rows.append(jnp.sqrt(jnp.sum(d * d, axis=1))[None, :])
    sim = jnp.concatenate(sim_rows, axis=0)
    cmax = jnp.max(sim, axis=1, keepdims=True)
    sim = sim / cmax
    sim_coding = jnp.mean(sim, axis=0, keepdims=True)

    h = lax.dot_general(sim_coding, wr_ref[...], dn,
                        preferred_element_type=jnp.float32) + br_ref[...][None, :]
    h = jnp.maximum(h, 0.0)
    bag = lax.dot_general(h, wc_ref[...], dn,
                          preferred_element_type=jnp.float32) + bc_ref[...][None, :]
    prob = jax.nn.softmax(bag, axis=1)

    bag_ref[...] = bag
    prob_ref[...] = prob
    yhat_ref[...] = jnp.where(prob[:, 1:2] > prob[:, 0:1], 1, 0).astype(jnp.int32)
    sim_ref[...] = sim_coding


def _tail(scores, x_path, prototype, W2, b2, Wr, br, Wc, bc):
    out_shapes = (
        jax.ShapeDtypeStruct((1, 2), jnp.float32),
        jax.ShapeDtypeStruct((1, 2), jnp.float32),
        jax.ShapeDtypeStruct((1, 1), jnp.int32),
        jax.ShapeDtypeStruct((1, K), jnp.float32),
    )
    vmem = lambda: pl.BlockSpec(memory_space=pltpu.MemorySpace.VMEM)
    return pl.pallas_call(
        _tail_body,
        in_specs=[
            vmem(),
            pl.BlockSpec(memory_space=pltpu.MemorySpace.HBM),
            vmem(), vmem(), vmem(), vmem(), vmem(), vmem(), vmem(),
        ],
        out_specs=(vmem(), vmem(), vmem(), vmem()),
        out_shape=out_shapes,
        scratch_shapes=[
            pltpu.VMEM((TOPK, D), jnp.float32),
            pltpu.SemaphoreType.DMA,
        ],
    )(scores, x_path, prototype, W2, b2, Wr, br, Wc, bc)


def kernel(x_path, prototype, W3, b3, W2, b2, Wr, br, Wc, bc):
    w = W3[1] - W3[0]
    scores = _scores(x_path, w)
    bag, prob, yhat, sim_coding = _tail(
        scores, x_path, prototype, W2, b2, Wr, br, Wc, bc)
    return (bag, prob, yhat.reshape(1), sim_coding)
